# baseline (speedup 1.0000x reference)
import functools

import jax
import jax.numpy as jnp
from jax import lax
from jax.experimental import pallas as pl
from jax.experimental.pallas import tpu as pltpu
from jax.experimental.pallas import tpu_sc as plsc

NUM_EFFECTS = 100
EMBED_DIM = 128
BATCH = 16384

_INFO = plsc.get_sparse_core_info()
_NC = _INFO.num_cores
_NS = _INFO.num_subcores
_NW = _NC * _NS
_B_PER_W = BATCH // _NW
_CHUNK = 128
_NCHUNK = _B_PER_W // _CHUNK

_mesh = plsc.VectorSubcoreMesh(core_axis_name="c", subcore_axis_name="s")


@functools.partial(
    pl.kernel,
    mesh=_mesh,
    out_type=jax.ShapeDtypeStruct((BATCH, EMBED_DIM), jnp.float32),
    scratch_types=[
        pltpu.VMEM((_NCHUNK, _CHUNK), jnp.int32),
        pltpu.VMEM((_B_PER_W, EMBED_DIM), jnp.float32),
        pltpu.VMEM_SHARED((NUM_EFFECTS, EMBED_DIM), jnp.float32),
    ]
    + [pltpu.SemaphoreType.DMA] * _NCHUNK
    + [pltpu.SemaphoreType.DMA, pltpu.SemaphoreType.DMA],
)
def _gather_kernel(idx_hbm, table_hbm, out_hbm, idx_v, rows_v, table_sh, *sems):
    gsems, ssem, isem = sems[:_NCHUNK], sems[_NCHUNK], sems[_NCHUNK + 1]
    s = lax.axis_index("s")
    wid = s * _NC + lax.axis_index("c")
    base = wid * _B_PER_W

    idx_cp = pltpu.async_copy(idx_hbm.at[wid], idx_v, isem)

    @pl.when(s == 0)
    def _load_table():
        pltpu.sync_copy(table_hbm, table_sh)

    plsc.subcore_barrier()
    idx_cp.wait()

    gathers = [
        pltpu.async_copy(
            (table_hbm if j == 0 else table_sh).at[idx_v.at[j]],
            rows_v.at[pl.ds(j * _CHUNK, _CHUNK)],
            gsems[j],
        )
        for j in range(_NCHUNK)
    ]
    stores = []
    for j in range(_NCHUNK):
        gathers[j].wait()
        stores.append(
            pltpu.async_copy(
                rows_v.at[pl.ds(j * _CHUNK, _CHUNK)],
                out_hbm.at[pl.ds(base + j * _CHUNK, _CHUNK)],
                ssem,
            )
        )
    for st in stores:
        st.wait()


def kernel(effect_id, embedding):
    idx = effect_id.reshape(_NW, _NCHUNK, _CHUNK)
    return _gather_kernel(idx, embedding)


# speedup vs baseline: 1.1738x; 1.1738x over previous
import functools

import jax
import jax.numpy as jnp
from jax import lax
from jax.experimental import pallas as pl
from jax.experimental.pallas import tpu as pltpu
from jax.experimental.pallas import tpu_sc as plsc

NUM_EFFECTS = 100
EMBED_DIM = 128
BATCH = 16384

_INFO = plsc.get_sparse_core_info()
_NC = _INFO.num_cores
_NS = _INFO.num_subcores
_NW = _NC * _NS
_B_PER_W = BATCH // _NW
_CHUNK = 128
_NCHUNK = _B_PER_W // _CHUNK

_mesh = plsc.VectorSubcoreMesh(core_axis_name="c", subcore_axis_name="s")


@functools.partial(
    pl.kernel,
    mesh=_mesh,
    out_type=jax.ShapeDtypeStruct((BATCH, EMBED_DIM), jnp.float32),
    scratch_types=[
        pltpu.VMEM((_NCHUNK, _CHUNK), jnp.int32),
        pltpu.VMEM((_B_PER_W, EMBED_DIM), jnp.float32),
        pltpu.VMEM_SHARED((NUM_EFFECTS, EMBED_DIM), jnp.float32),
    ]
    + [pltpu.SemaphoreType.DMA] * _NCHUNK
    + [pltpu.SemaphoreType.DMA, pltpu.SemaphoreType.DMA],
)
def _gather_kernel(idx_hbm, table_hbm, out_hbm, idx_v, rows_v, table_sh, *sems):
    gsems, ssem, isem = sems[:_NCHUNK], sems[_NCHUNK], sems[_NCHUNK + 1]
    s = lax.axis_index("s")
    wid = s * _NC + lax.axis_index("c")
    base = wid * _B_PER_W

    idx_cp = pltpu.async_copy(idx_hbm.at[wid], idx_v, isem)

    @pl.when(s < 3)
    def _load_table():
        pltpu.sync_copy(
            table_hbm.at[pl.ds(s * 32, 32)], table_sh.at[pl.ds(s * 32, 32)]
        )

    @pl.when(s == 3)
    def _load_table_tail():
        pltpu.sync_copy(
            table_hbm.at[pl.ds(96, NUM_EFFECTS - 96)],
            table_sh.at[pl.ds(96, NUM_EFFECTS - 96)],
        )

    plsc.subcore_barrier()
    idx_cp.wait()

    gathers = [
        pltpu.async_copy(
            table_sh.at[idx_v.at[j]],
            rows_v.at[pl.ds(j * _CHUNK, _CHUNK)],
            gsems[j],
        )
        for j in range(_NCHUNK)
    ]
    stores = []
    for j in range(_NCHUNK):
        gathers[j].wait()
        stores.append(
            pltpu.async_copy(
                rows_v.at[pl.ds(j * _CHUNK, _CHUNK)],
                out_hbm.at[pl.ds(base + j * _CHUNK, _CHUNK)],
                ssem,
            )
        )
    for st in stores:
        st.wait()


def kernel(effect_id, embedding):
    idx = effect_id.reshape(_NW, _NCHUNK, _CHUNK)
    return _gather_kernel(idx, embedding)
